# baseline (device time: 362681 ns/iter reference)
import jax
import jax.numpy as jnp
from jax import lax
from jax.experimental import pallas as pl
from jax.experimental.pallas import tpu as pltpu

N_DEV = 32
B = 2
SQ = 512
SKV = 512
H_PER = 8
DH = 64
D_MODEL = 768
ROWS = B * SQ
CH = ROWS // N_DEV


def _grp_rows(a):
    n = a.shape[-1]
    return a.reshape(2, 4, 64, n).transpose(1, 0, 2, 3).reshape(SQ, n)


def _ungrp_rows(a):
    n = a.shape[-1]
    return a.reshape(4, 2, 64, n).transpose(1, 0, 2, 3).reshape(SQ, n)


def _attn_body(m_ref, x_ref, wq_ref, k_ref, v_ref, wo_ref, out_ref):
    del m_ref
    xg = _grp_rows(x_ref[0])
    q = jnp.dot(xg, wq_ref[...], preferred_element_type=jnp.float32)
    kf = _grp_rows(k_ref[0])
    vf = _grp_rows(v_ref[0])
    ctxs = []
    for h in range(H_PER):
        sl = slice(h * DH, (h + 1) * DH)
        qh = q[:, sl].reshape(4, 128, DH)
        kh = kf[:, sl].reshape(4, 128, DH)
        vh = vf[:, sl].reshape(4, 128, DH)
        s = lax.dot_general(qh, kh, (((2,), (2,)), ((0,), (0,))),
                            preferred_element_type=jnp.float32) * 0.125
        m = jnp.max(s, axis=-1, keepdims=True)
        w = jnp.exp(s - m)
        w = w / jnp.sum(w, axis=-1, keepdims=True)
        ch = lax.dot_general(w, vh, (((2,), (1,)), ((0,), (0,))),
                             preferred_element_type=jnp.float32)
        ctxs.append(ch.reshape(SQ, DH))
    ctx = _ungrp_rows(jnp.concatenate(ctxs, axis=1))
    out_ref[0] = jnp.dot(ctx, wo_ref[...], preferred_element_type=jnp.float32)


def _ar_body(p_ref, out_ref, acc_ref, send1, recv1, send2, recv2):
    my = lax.axis_index("i")

    p1 = []
    for p in range(N_DEV):
        r = pltpu.make_async_remote_copy(
            src_ref=p_ref.at[pl.ds(p * CH, CH)],
            dst_ref=acc_ref.at[pl.ds(my * CH, CH)],
            send_sem=send1.at[p],
            recv_sem=recv1.at[my],
            device_id=(p,),
            device_id_type=pl.DeviceIdType.MESH,
        )
        p1.append(r)

        @pl.when(my != p)
        def _(r=r):
            r.start()

    for s in range(N_DEV):
        rcv = pltpu.make_async_remote_copy(
            src_ref=acc_ref.at[pl.ds(s * CH, CH)],
            dst_ref=acc_ref.at[pl.ds(s * CH, CH)],
            send_sem=send1.at[s],
            recv_sem=recv1.at[s],
            device_id=(s,),
            device_id_type=pl.DeviceIdType.MESH,
        )

        @pl.when(my != s)
        def _(rcv=rcv):
            rcv.wait_recv()

    own = p_ref[pl.ds(my * CH, CH), :]
    acc = acc_ref[...].reshape(N_DEV, CH, D_MODEL)
    sidx = lax.broadcasted_iota(jnp.int32, (N_DEV, 1, 1), 0)
    red = own + jnp.sum(jnp.where(sidx == my, 0.0, acc), axis=0)
    out_ref[pl.ds(my * CH, CH), :] = red

    p2 = []
    for p in range(N_DEV):
        r = pltpu.make_async_remote_copy(
            src_ref=out_ref.at[pl.ds(my * CH, CH)],
            dst_ref=out_ref.at[pl.ds(my * CH, CH)],
            send_sem=send2.at[p],
            recv_sem=recv2.at[my],
            device_id=(p,),
            device_id_type=pl.DeviceIdType.MESH,
        )
        p2.append(r)

        @pl.when(my != p)
        def _(r=r):
            r.start()

    for s in range(N_DEV):
        rcv = pltpu.make_async_remote_copy(
            src_ref=out_ref.at[pl.ds(s * CH, CH)],
            dst_ref=out_ref.at[pl.ds(s * CH, CH)],
            send_sem=send2.at[s],
            recv_sem=recv2.at[s],
            device_id=(s,),
            device_id_type=pl.DeviceIdType.MESH,
        )

        @pl.when(my != s)
        def _(rcv=rcv):
            rcv.wait_recv()

    for p in range(N_DEV):
        @pl.when(my != p)
        def _(r=p1[p]):
            r.wait_send()

        @pl.when(my != p)
        def _(r=p2[p]):
            r.wait_send()


def kernel(x, Wq, K_ext, V_ext, Wo):
    my = lax.axis_index("i")
    midx = jnp.reshape(my, (1,)).astype(jnp.int32)

    HD = H_PER * DH
    partial = pl.pallas_call(
        _attn_body,
        grid_spec=pltpu.PrefetchScalarGridSpec(
            num_scalar_prefetch=1,
            grid=(B,),
            in_specs=[
                pl.BlockSpec((1, SQ, D_MODEL), lambda b, m: (b, 0, 0)),
                pl.BlockSpec((D_MODEL, HD), lambda b, m: (0, 0)),
                pl.BlockSpec((1, SKV, HD), lambda b, m: (b, 0, m[0])),
                pl.BlockSpec((1, SKV, HD), lambda b, m: (b, 0, m[0])),
                pl.BlockSpec((HD, D_MODEL), lambda b, m: (0, 0)),
            ],
            out_specs=pl.BlockSpec((1, SQ, D_MODEL), lambda b, m: (b, 0, 0)),
        ),
        out_shape=jax.ShapeDtypeStruct((B, SQ, D_MODEL), jnp.float32),
    )(midx, x, Wq, K_ext.reshape(B, SKV, 256 * DH), V_ext.reshape(B, SKV, 256 * DH), Wo)

    out = pl.pallas_call(
        _ar_body,
        out_shape=jax.ShapeDtypeStruct((ROWS, D_MODEL), jnp.float32),
        in_specs=[pl.BlockSpec(memory_space=pltpu.VMEM)],
        out_specs=pl.BlockSpec(memory_space=pltpu.VMEM),
        scratch_shapes=[
            pltpu.VMEM((ROWS, D_MODEL), jnp.float32),
            pltpu.SemaphoreType.DMA((N_DEV,)),
            pltpu.SemaphoreType.DMA((N_DEV,)),
            pltpu.SemaphoreType.DMA((N_DEV,)),
            pltpu.SemaphoreType.DMA((N_DEV,)),
        ],
    )(partial.reshape(ROWS, D_MODEL))
    return out.reshape(B, SQ, D_MODEL)


# device time: 300040 ns/iter; 1.2088x vs baseline; 1.2088x over previous
import jax
import jax.numpy as jnp
from jax import lax
from jax.experimental import pallas as pl
from jax.experimental.pallas import tpu as pltpu

N_DEV = 32
B = 2
SQ = 512
SKV = 512
H_PER = 8
DH = 64
D_MODEL = 768
ROWS = B * SQ
CH = ROWS // N_DEV


def _grp_rows(a):
    n = a.shape[-1]
    return a.reshape(2, 4, 64, n).transpose(1, 0, 2, 3).reshape(SQ, n)


def _ungrp_rows(a):
    n = a.shape[-1]
    return a.reshape(4, 2, 64, n).transpose(1, 0, 2, 3).reshape(SQ, n)


def _attn_body(x_ref, wq_ref, k_ref, v_ref, wo_ref, out_ref,
               kbuf, vbuf, ksems, vsems):
    my = lax.axis_index("i")
    b = pl.program_id(0)

    for h in range(H_PER):
        pltpu.make_async_copy(
            k_ref.at[b, :, my * H_PER + h, :], kbuf.at[h], ksems.at[h]
        ).start()
        pltpu.make_async_copy(
            v_ref.at[b, :, my * H_PER + h, :], vbuf.at[h], vsems.at[h]
        ).start()

    xg = _grp_rows(x_ref[0])
    q = jnp.dot(xg, wq_ref[...], preferred_element_type=jnp.float32)
    ctxs = []
    for h in range(H_PER):
        pltpu.make_async_copy(
            k_ref.at[b, :, my * H_PER + h, :], kbuf.at[h], ksems.at[h]
        ).wait()
        pltpu.make_async_copy(
            v_ref.at[b, :, my * H_PER + h, :], vbuf.at[h], vsems.at[h]
        ).wait()
        qh = q[:, h * DH:(h + 1) * DH].reshape(4, 128, DH)
        kh = _grp_rows(kbuf[h]).reshape(4, 128, DH)
        vh = _grp_rows(vbuf[h]).reshape(4, 128, DH)
        s = lax.dot_general(qh, kh, (((2,), (2,)), ((0,), (0,))),
                            preferred_element_type=jnp.float32) * 0.125
        m = jnp.max(s, axis=-1, keepdims=True)
        w = jnp.exp(s - m)
        w = w / jnp.sum(w, axis=-1, keepdims=True)
        ch = lax.dot_general(w, vh, (((2,), (1,)), ((0,), (0,))),
                             preferred_element_type=jnp.float32)
        ctxs.append(ch.reshape(SQ, DH))
    ctx = _ungrp_rows(jnp.concatenate(ctxs, axis=1))
    out_ref[0] = jnp.dot(ctx, wo_ref[...], preferred_element_type=jnp.float32)


def _ar_body(p_ref, out_ref, acc_ref, send1, recv1, send2, recv2):
    my = lax.axis_index("i")

    p1 = []
    for p in range(N_DEV):
        r = pltpu.make_async_remote_copy(
            src_ref=p_ref.at[pl.ds(p * CH, CH)],
            dst_ref=acc_ref.at[pl.ds(my * CH, CH)],
            send_sem=send1.at[p],
            recv_sem=recv1.at[my],
            device_id=(p,),
            device_id_type=pl.DeviceIdType.MESH,
        )
        p1.append(r)

        @pl.when(my != p)
        def _(r=r):
            r.start()

    for s in range(N_DEV):
        rcv = pltpu.make_async_remote_copy(
            src_ref=acc_ref.at[pl.ds(s * CH, CH)],
            dst_ref=acc_ref.at[pl.ds(s * CH, CH)],
            send_sem=send1.at[s],
            recv_sem=recv1.at[s],
            device_id=(s,),
            device_id_type=pl.DeviceIdType.MESH,
        )

        @pl.when(my != s)
        def _(rcv=rcv):
            rcv.wait_recv()

    own = p_ref[pl.ds(my * CH, CH), :]
    acc = acc_ref[...].reshape(N_DEV, CH, D_MODEL)
    sidx = lax.broadcasted_iota(jnp.int32, (N_DEV, 1, 1), 0)
    red = own + jnp.sum(jnp.where(sidx == my, 0.0, acc), axis=0)
    out_ref[pl.ds(my * CH, CH), :] = red

    p2 = []
    for p in range(N_DEV):
        r = pltpu.make_async_remote_copy(
            src_ref=out_ref.at[pl.ds(my * CH, CH)],
            dst_ref=out_ref.at[pl.ds(my * CH, CH)],
            send_sem=send2.at[p],
            recv_sem=recv2.at[my],
            device_id=(p,),
            device_id_type=pl.DeviceIdType.MESH,
        )
        p2.append(r)

        @pl.when(my != p)
        def _(r=r):
            r.start()

    for s in range(N_DEV):
        rcv = pltpu.make_async_remote_copy(
            src_ref=out_ref.at[pl.ds(s * CH, CH)],
            dst_ref=out_ref.at[pl.ds(s * CH, CH)],
            send_sem=send2.at[s],
            recv_sem=recv2.at[s],
            device_id=(s,),
            device_id_type=pl.DeviceIdType.MESH,
        )

        @pl.when(my != s)
        def _(rcv=rcv):
            rcv.wait_recv()

    for p in range(N_DEV):
        @pl.when(my != p)
        def _(r=p1[p]):
            r.wait_send()

        @pl.when(my != p)
        def _(r=p2[p]):
            r.wait_send()


def kernel(x, Wq, K_ext, V_ext, Wo):
    HD = H_PER * DH
    partial = pl.pallas_call(
        _attn_body,
        grid=(B,),
        in_specs=[
            pl.BlockSpec((1, SQ, D_MODEL), lambda b: (b, 0, 0)),
            pl.BlockSpec((D_MODEL, HD), lambda b: (0, 0)),
            pl.BlockSpec(memory_space=pltpu.MemorySpace.HBM),
            pl.BlockSpec(memory_space=pltpu.MemorySpace.HBM),
            pl.BlockSpec((HD, D_MODEL), lambda b: (0, 0)),
        ],
        out_specs=pl.BlockSpec((1, SQ, D_MODEL), lambda b: (b, 0, 0)),
        out_shape=jax.ShapeDtypeStruct((B, SQ, D_MODEL), jnp.float32),
        scratch_shapes=[
            pltpu.VMEM((H_PER, SKV, DH), jnp.float32),
            pltpu.VMEM((H_PER, SKV, DH), jnp.float32),
            pltpu.SemaphoreType.DMA((H_PER,)),
            pltpu.SemaphoreType.DMA((H_PER,)),
        ],
    )(x, Wq, K_ext, V_ext, Wo)

    out = pl.pallas_call(
        _ar_body,
        out_shape=jax.ShapeDtypeStruct((ROWS, D_MODEL), jnp.float32),
        in_specs=[pl.BlockSpec(memory_space=pltpu.VMEM)],
        out_specs=pl.BlockSpec(memory_space=pltpu.VMEM),
        scratch_shapes=[
            pltpu.VMEM((ROWS, D_MODEL), jnp.float32),
            pltpu.SemaphoreType.DMA((N_DEV,)),
            pltpu.SemaphoreType.DMA((N_DEV,)),
            pltpu.SemaphoreType.DMA((N_DEV,)),
            pltpu.SemaphoreType.DMA((N_DEV,)),
        ],
    )(partial.reshape(ROWS, D_MODEL))
    return out.reshape(B, SQ, D_MODEL)


# device time: 138735 ns/iter; 2.6142x vs baseline; 2.1627x over previous
import jax
import jax.numpy as jnp
from jax import lax
from jax.experimental import pallas as pl
from jax.experimental.pallas import tpu as pltpu

N_DEV = 32
B = 2
SQ = 512
SKV = 512
H_PER = 8
DH = 64
D_MODEL = 768
ROWS = B * SQ
CH = ROWS // N_DEV


def _grp_rows(a):
    n = a.shape[-1]
    return a.reshape(2, 4, 64, n).transpose(1, 0, 2, 3).reshape(SQ, n)


def _ungrp_rows(a):
    n = a.shape[-1]
    return a.reshape(4, 2, 64, n).transpose(1, 0, 2, 3).reshape(SQ, n)


_SB = 128


def _extract_body(m_ref, k_ref, v_ref, ko_ref, vo_ref):
    del m_ref
    my = lax.axis_index("i")
    off = (my % 16) * H_PER
    li = lax.broadcasted_iota(jnp.int32, (128, H_PER), 0)
    lj = lax.broadcasted_iota(jnp.int32, (128, H_PER), 1)
    sel = (li == off + lj).astype(jnp.float32)

    def extract(ref):
        a8 = jnp.dot(ref[0].reshape(_SB * DH, 128), sel,
                     preferred_element_type=jnp.float32)
        return jnp.swapaxes(a8.reshape(_SB, DH, H_PER), 1, 2).reshape(
            _SB, H_PER * DH)

    ko_ref[0] = extract(k_ref)
    vo_ref[0] = extract(v_ref)


def _attn_body(x_ref, wq_ref, k_ref, v_ref, wo_ref, out_ref):
    xg = _grp_rows(x_ref[0])
    q = jnp.dot(xg, wq_ref[...], preferred_element_type=jnp.float32)
    kf = _grp_rows(k_ref[0])
    vf = _grp_rows(v_ref[0])
    ctxs = []
    for h in range(H_PER):
        sl = slice(h * DH, (h + 1) * DH)
        qh = q[:, sl].reshape(4, 128, DH)
        kh = kf[:, sl].reshape(4, 128, DH)
        vh = vf[:, sl].reshape(4, 128, DH)
        s = lax.dot_general(qh, kh, (((2,), (2,)), ((0,), (0,))),
                            preferred_element_type=jnp.float32) * 0.125
        m = jnp.max(s, axis=-1, keepdims=True)
        w = jnp.exp(s - m)
        w = w / jnp.sum(w, axis=-1, keepdims=True)
        ch = lax.dot_general(w, vh, (((2,), (1,)), ((0,), (0,))),
                             preferred_element_type=jnp.float32)
        ctxs.append(ch.reshape(SQ, DH))
    ctx = _ungrp_rows(jnp.concatenate(ctxs, axis=1))
    out_ref[0] = jnp.dot(ctx, wo_ref[...], preferred_element_type=jnp.float32)


def _ar_body(p_ref, out_ref, acc_ref, send1, recv1, send2, recv2):
    my = lax.axis_index("i")

    p1 = []
    for p in range(N_DEV):
        r = pltpu.make_async_remote_copy(
            src_ref=p_ref.at[pl.ds(p * CH, CH)],
            dst_ref=acc_ref.at[pl.ds(my * CH, CH)],
            send_sem=send1.at[p],
            recv_sem=recv1.at[my],
            device_id=(p,),
            device_id_type=pl.DeviceIdType.MESH,
        )
        p1.append(r)

        @pl.when(my != p)
        def _(r=r):
            r.start()

    for s in range(N_DEV):
        rcv = pltpu.make_async_remote_copy(
            src_ref=acc_ref.at[pl.ds(s * CH, CH)],
            dst_ref=acc_ref.at[pl.ds(s * CH, CH)],
            send_sem=send1.at[s],
            recv_sem=recv1.at[s],
            device_id=(s,),
            device_id_type=pl.DeviceIdType.MESH,
        )

        @pl.when(my != s)
        def _(rcv=rcv):
            rcv.wait_recv()

    own = p_ref[pl.ds(my * CH, CH), :]
    acc = acc_ref[...].reshape(N_DEV, CH, D_MODEL)
    sidx = lax.broadcasted_iota(jnp.int32, (N_DEV, 1, 1), 0)
    red = own + jnp.sum(jnp.where(sidx == my, 0.0, acc), axis=0)
    out_ref[pl.ds(my * CH, CH), :] = red

    p2 = []
    for p in range(N_DEV):
        r = pltpu.make_async_remote_copy(
            src_ref=out_ref.at[pl.ds(my * CH, CH)],
            dst_ref=out_ref.at[pl.ds(my * CH, CH)],
            send_sem=send2.at[p],
            recv_sem=recv2.at[my],
            device_id=(p,),
            device_id_type=pl.DeviceIdType.MESH,
        )
        p2.append(r)

        @pl.when(my != p)
        def _(r=r):
            r.start()

    for s in range(N_DEV):
        rcv = pltpu.make_async_remote_copy(
            src_ref=out_ref.at[pl.ds(s * CH, CH)],
            dst_ref=out_ref.at[pl.ds(s * CH, CH)],
            send_sem=send2.at[s],
            recv_sem=recv2.at[s],
            device_id=(s,),
            device_id_type=pl.DeviceIdType.MESH,
        )

        @pl.when(my != s)
        def _(rcv=rcv):
            rcv.wait_recv()

    for p in range(N_DEV):
        @pl.when(my != p)
        def _(r=p1[p]):
            r.wait_send()

        @pl.when(my != p)
        def _(r=p2[p]):
            r.wait_send()


def kernel(x, Wq, K_ext, V_ext, Wo):
    HD = H_PER * DH
    my = lax.axis_index("i")
    midx = jnp.reshape(my // 16, (1,)).astype(jnp.int32)
    Kt = jnp.transpose(K_ext, (0, 1, 3, 2))
    Vt = jnp.transpose(V_ext, (0, 1, 3, 2))
    kflat, vflat = pl.pallas_call(
        _extract_body,
        grid_spec=pltpu.PrefetchScalarGridSpec(
            num_scalar_prefetch=1,
            grid=(B, SKV // _SB),
            in_specs=[
                pl.BlockSpec((1, _SB, DH, 128), lambda b, s, m: (b, s, 0, m[0])),
                pl.BlockSpec((1, _SB, DH, 128), lambda b, s, m: (b, s, 0, m[0])),
            ],
            out_specs=[
                pl.BlockSpec((1, _SB, HD), lambda b, s, m: (b, s, 0)),
                pl.BlockSpec((1, _SB, HD), lambda b, s, m: (b, s, 0)),
            ],
        ),
        out_shape=[
            jax.ShapeDtypeStruct((B, SKV, HD), jnp.float32),
            jax.ShapeDtypeStruct((B, SKV, HD), jnp.float32),
        ],
    )(midx, Kt, Vt)

    partial = pl.pallas_call(
        _attn_body,
        grid=(B,),
        in_specs=[
            pl.BlockSpec((1, SQ, D_MODEL), lambda b: (b, 0, 0)),
            pl.BlockSpec((D_MODEL, HD), lambda b: (0, 0)),
            pl.BlockSpec((1, SKV, HD), lambda b: (b, 0, 0)),
            pl.BlockSpec((1, SKV, HD), lambda b: (b, 0, 0)),
            pl.BlockSpec((HD, D_MODEL), lambda b: (0, 0)),
        ],
        out_specs=pl.BlockSpec((1, SQ, D_MODEL), lambda b: (b, 0, 0)),
        out_shape=jax.ShapeDtypeStruct((B, SQ, D_MODEL), jnp.float32),
    )(x, Wq, kflat, vflat, Wo)

    out = pl.pallas_call(
        _ar_body,
        out_shape=jax.ShapeDtypeStruct((ROWS, D_MODEL), jnp.float32),
        in_specs=[pl.BlockSpec(memory_space=pltpu.VMEM)],
        out_specs=pl.BlockSpec(memory_space=pltpu.VMEM),
        scratch_shapes=[
            pltpu.VMEM((ROWS, D_MODEL), jnp.float32),
            pltpu.SemaphoreType.DMA((N_DEV,)),
            pltpu.SemaphoreType.DMA((N_DEV,)),
            pltpu.SemaphoreType.DMA((N_DEV,)),
            pltpu.SemaphoreType.DMA((N_DEV,)),
        ],
    )(partial.reshape(ROWS, D_MODEL))
    return out.reshape(B, SQ, D_MODEL)


# device time: 136668 ns/iter; 2.6537x vs baseline; 1.0151x over previous
import jax
import jax.numpy as jnp
from jax import lax
from jax.experimental import pallas as pl
from jax.experimental.pallas import tpu as pltpu

N_DEV = 32
B = 2
SQ = 512
SKV = 512
H_PER = 8
DH = 64
D_MODEL = 768
ROWS = B * SQ
CH = ROWS // N_DEV


def _grp_rows(a):
    n = a.shape[-1]
    return a.reshape(2, 4, 64, n).transpose(1, 0, 2, 3).reshape(SQ, n)


def _ungrp_rows(a):
    n = a.shape[-1]
    return a.reshape(4, 2, 64, n).transpose(1, 0, 2, 3).reshape(SQ, n)


_SB = 128


def _extract_body(m_ref, k_ref, v_ref, ko_ref, vo_ref):
    del m_ref
    my = lax.axis_index("i")
    off = (my % 16) * H_PER
    li = lax.broadcasted_iota(jnp.int32, (128, H_PER), 0)
    lj = lax.broadcasted_iota(jnp.int32, (128, H_PER), 1)
    sel = (li == off + lj).astype(jnp.float32)

    def extract(ref):
        a8 = jnp.dot(ref[0].reshape(_SB * DH, 128), sel,
                     preferred_element_type=jnp.float32)
        return jnp.swapaxes(a8.reshape(_SB, DH, H_PER), 1, 2).reshape(
            _SB, H_PER * DH)

    ko_ref[0] = extract(k_ref)
    vo_ref[0] = extract(v_ref)


def _attn_body(x_ref, wq_ref, k_ref, v_ref, wo_ref, out_ref):
    bf = jnp.bfloat16
    xg = _grp_rows(x_ref[0]).astype(bf)
    q = jnp.dot(xg, wq_ref[...].astype(bf),
                preferred_element_type=jnp.float32).astype(bf)
    kf = _grp_rows(k_ref[0]).astype(bf)
    vf = _grp_rows(v_ref[0]).astype(bf)
    ctxs = []
    for h in range(H_PER):
        sl = slice(h * DH, (h + 1) * DH)
        qh = q[:, sl].reshape(4, 128, DH)
        kh = kf[:, sl].reshape(4, 128, DH)
        vh = vf[:, sl].reshape(4, 128, DH)
        s = lax.dot_general(qh, kh, (((2,), (2,)), ((0,), (0,))),
                            preferred_element_type=jnp.float32) * 0.125
        m = jnp.max(s, axis=-1, keepdims=True)
        w = jnp.exp(s - m)
        w = (w / jnp.sum(w, axis=-1, keepdims=True)).astype(bf)
        ch = lax.dot_general(w, vh, (((2,), (1,)), ((0,), (0,))),
                             preferred_element_type=jnp.float32)
        ctxs.append(ch.astype(bf).reshape(SQ, DH))
    ctx = _ungrp_rows(jnp.concatenate(ctxs, axis=1))
    out_ref[0] = jnp.dot(ctx, wo_ref[...].astype(bf),
                         preferred_element_type=jnp.float32)


def _ar_body(p_ref, out_ref, acc_ref, send1, recv1, send2, recv2):
    my = lax.axis_index("i")

    p1 = []
    for p in range(N_DEV):
        r = pltpu.make_async_remote_copy(
            src_ref=p_ref.at[pl.ds(p * CH, CH)],
            dst_ref=acc_ref.at[pl.ds(my * CH, CH)],
            send_sem=send1.at[p],
            recv_sem=recv1.at[my],
            device_id=(p,),
            device_id_type=pl.DeviceIdType.MESH,
        )
        p1.append(r)

        @pl.when(my != p)
        def _(r=r):
            r.start()

    for s in range(N_DEV):
        rcv = pltpu.make_async_remote_copy(
            src_ref=acc_ref.at[pl.ds(s * CH, CH)],
            dst_ref=acc_ref.at[pl.ds(s * CH, CH)],
            send_sem=send1.at[s],
            recv_sem=recv1.at[s],
            device_id=(s,),
            device_id_type=pl.DeviceIdType.MESH,
        )

        @pl.when(my != s)
        def _(rcv=rcv):
            rcv.wait_recv()

    own = p_ref[pl.ds(my * CH, CH), :]
    acc = acc_ref[...].reshape(N_DEV, CH, D_MODEL)
    sidx = lax.broadcasted_iota(jnp.int32, (N_DEV, 1, 1), 0)
    red = own + jnp.sum(jnp.where(sidx == my, 0.0, acc), axis=0)
    out_ref[pl.ds(my * CH, CH), :] = red

    p2 = []
    for p in range(N_DEV):
        r = pltpu.make_async_remote_copy(
            src_ref=out_ref.at[pl.ds(my * CH, CH)],
            dst_ref=out_ref.at[pl.ds(my * CH, CH)],
            send_sem=send2.at[p],
            recv_sem=recv2.at[my],
            device_id=(p,),
            device_id_type=pl.DeviceIdType.MESH,
        )
        p2.append(r)

        @pl.when(my != p)
        def _(r=r):
            r.start()

    for s in range(N_DEV):
        rcv = pltpu.make_async_remote_copy(
            src_ref=out_ref.at[pl.ds(s * CH, CH)],
            dst_ref=out_ref.at[pl.ds(s * CH, CH)],
            send_sem=send2.at[s],
            recv_sem=recv2.at[s],
            device_id=(s,),
            device_id_type=pl.DeviceIdType.MESH,
        )

        @pl.when(my != s)
        def _(rcv=rcv):
            rcv.wait_recv()

    for p in range(N_DEV):
        @pl.when(my != p)
        def _(r=p1[p]):
            r.wait_send()

        @pl.when(my != p)
        def _(r=p2[p]):
            r.wait_send()


def kernel(x, Wq, K_ext, V_ext, Wo):
    HD = H_PER * DH
    my = lax.axis_index("i")
    midx = jnp.reshape(my // 16, (1,)).astype(jnp.int32)
    Kt = jnp.transpose(K_ext, (0, 1, 3, 2))
    Vt = jnp.transpose(V_ext, (0, 1, 3, 2))
    kflat, vflat = pl.pallas_call(
        _extract_body,
        grid_spec=pltpu.PrefetchScalarGridSpec(
            num_scalar_prefetch=1,
            grid=(B, SKV // _SB),
            in_specs=[
                pl.BlockSpec((1, _SB, DH, 128), lambda b, s, m: (b, s, 0, m[0])),
                pl.BlockSpec((1, _SB, DH, 128), lambda b, s, m: (b, s, 0, m[0])),
            ],
            out_specs=[
                pl.BlockSpec((1, _SB, HD), lambda b, s, m: (b, s, 0)),
                pl.BlockSpec((1, _SB, HD), lambda b, s, m: (b, s, 0)),
            ],
        ),
        out_shape=[
            jax.ShapeDtypeStruct((B, SKV, HD), jnp.float32),
            jax.ShapeDtypeStruct((B, SKV, HD), jnp.float32),
        ],
    )(midx, Kt, Vt)

    partial = pl.pallas_call(
        _attn_body,
        grid=(B,),
        in_specs=[
            pl.BlockSpec((1, SQ, D_MODEL), lambda b: (b, 0, 0)),
            pl.BlockSpec((D_MODEL, HD), lambda b: (0, 0)),
            pl.BlockSpec((1, SKV, HD), lambda b: (b, 0, 0)),
            pl.BlockSpec((1, SKV, HD), lambda b: (b, 0, 0)),
            pl.BlockSpec((HD, D_MODEL), lambda b: (0, 0)),
        ],
        out_specs=pl.BlockSpec((1, SQ, D_MODEL), lambda b: (b, 0, 0)),
        out_shape=jax.ShapeDtypeStruct((B, SQ, D_MODEL), jnp.float32),
    )(x, Wq, kflat, vflat, Wo)

    out = pl.pallas_call(
        _ar_body,
        out_shape=jax.ShapeDtypeStruct((ROWS, D_MODEL), jnp.float32),
        in_specs=[pl.BlockSpec(memory_space=pltpu.VMEM)],
        out_specs=pl.BlockSpec(memory_space=pltpu.VMEM),
        scratch_shapes=[
            pltpu.VMEM((ROWS, D_MODEL), jnp.float32),
            pltpu.SemaphoreType.DMA((N_DEV,)),
            pltpu.SemaphoreType.DMA((N_DEV,)),
            pltpu.SemaphoreType.DMA((N_DEV,)),
            pltpu.SemaphoreType.DMA((N_DEV,)),
        ],
    )(partial.reshape(ROWS, D_MODEL))
    return out.reshape(B, SQ, D_MODEL)
